# Initial kernel scaffold; baseline (speedup 1.0000x reference)
import jax
import jax.numpy as jnp
from jax.experimental import pallas as pl


def kernel(output, target):
    raise NotImplementedError("write your pallas kernel here")


# speedup vs baseline: 25.3995x; 25.3995x over previous
import functools

import numpy as np
import jax
import jax.numpy as jnp
from jax import lax
from jax.experimental import pallas as pl
from jax.experimental.pallas import tpu as pltpu
from jax.experimental.pallas import tpu_sc as plsc

R = 2048
N = 16384
NB = 4096
HPAD = NB + 1
LO, HI = -8.0, 8.0
SCALE = NB / (HI - LO)
NVEC = N // 16
NBLK = NB // 16
EPS = 1e-5
C_DEV = float(0.02 * 2.0 * 0.7978845608028654)

_M1 = np.uint32(0x85EBCA6B)
_M2 = np.uint32(0xC2B2AE35)


def _make_sc_kernel():
    info = plsc.get_sparse_core_info()
    nc, ns = info.num_cores, info.num_subcores
    nw = nc * ns
    rows_per = R // nw
    mesh = plsc.VectorSubcoreMesh(core_axis_name="c", subcore_axis_name="s")

    @functools.partial(
        pl.kernel,
        mesh=mesh,
        compiler_params=pltpu.CompilerParams(needs_layout_passes=False),
        out_type=jax.ShapeDtypeStruct((R,), jnp.float32),
        scratch_types=[
            pltpu.VMEM((N,), jnp.float32),
            pltpu.VMEM((N,), jnp.int32),
            pltpu.VMEM((16 * HPAD,), jnp.int32),
            pltpu.VMEM((NB,), jnp.int32),
            pltpu.VMEM((NB,), jnp.int32),
            pltpu.VMEM((NBLK,), jnp.int32),
            pltpu.VMEM((rows_per,), jnp.float32),
        ],
    )
    def sc_ap(x_hbm, t_hbm, ap_hbm, sbuf, tbuf, hist, scan, cnt, ebp, apbuf):
        wid = lax.axis_index("s") * nc + lax.axis_index("c")
        iota16 = lax.iota(jnp.int32, 16)
        iota16u = lax.iota(jnp.uint32, 16)
        zeros16 = jnp.zeros((16,), jnp.int32)

        lane_off = iota16 * HPAD

        def z_body(i, c):
            for l in range(16):
                hist[pl.ds(i * 16 + l * HPAD, 16)] = zeros16
            return c
        lax.fori_loop(0, NBLK, z_body, 0)

        def row_body(lr, c):
            row = wid * rows_per + lr
            pltpu.sync_copy(x_hbm.at[row], sbuf)
            pltpu.sync_copy(t_hbm.at[row], tbuf)
            base0 = (row * N).astype(jnp.uint32)

            def p1(i, c1):
                sl = pl.ds(i * 16, 16)
                s = sbuf[sl]
                t = tbuf[sl]
                h = base0 + (i * 16).astype(jnp.uint32) + iota16u
                h = h ^ (h >> jnp.uint32(16))
                h = h * _M1
                h = h ^ (h >> jnp.uint32(13))
                h = h * _M2
                h = h ^ (h >> jnp.uint32(16))
                u = (h & jnp.uint32(0x00FFFFFF)).astype(jnp.float32)
                tf = t.astype(jnp.float32)
                sc = s - (u * jnp.float32(C_DEV * 2.0 ** -24)) * (tf - 0.5)
                xb = (jnp.float32(HI) - sc) * jnp.float32(SCALE)
                xb = jnp.minimum(jnp.maximum(xb, jnp.float32(0.0)),
                                 jnp.float32(NB - 1))
                b = xb.astype(jnp.int32)
                tt = t << 16
                plsc.addupdate_scatter(hist, [lane_off + b], jnp.int32(1) + tt)
                sbuf[sl] = plsc.bitcast(b + tt, jnp.float32)
                return c1
            lax.fori_loop(0, NVEC, p1, 0)

            def l1(i, c1):
                sl = pl.ds(i * 16, 16)
                v = hist[pl.ds(i * 16, 16)]
                hist[pl.ds(i * 16, 16)] = zeros16
                for l in range(1, 16):
                    hsl = pl.ds(i * 16 + l * HPAD, 16)
                    v = v + hist[hsl]
                    hist[hsl] = zeros16
                cnt[sl] = v
                scan[sl] = plsc.cumsum(v)
                return c1
            lax.fori_loop(0, NBLK, l1, 0)

            idx_be = iota16 * 16 + 15
            def l2(j, carry):
                be = plsc.load_gather(scan, [j * 256 + idx_be])
                cs = plsc.cumsum(be)
                ebp[pl.ds(j * 16, 16)] = carry + cs - be
                return carry + jnp.sum(be)
            tot = lax.fori_loop(0, 16, l2, jnp.int32(0))
            pcount = jnp.right_shift(tot, 16)

            def p2(i, acc):
                sl = pl.ds(i * 16, 16)
                pk = plsc.bitcast(sbuf[sl], jnp.int32)
                b = pk & jnp.int32(0xFFFF)
                t = jnp.right_shift(pk, 16)
                inc = (plsc.load_gather(scan, [b])
                       + plsc.load_gather(ebp, [jnp.right_shift(b, 4)]))
                cv = plsc.load_gather(cnt, [b])
                nlo = cv & jnp.int32(0xFFFF)
                khi = jnp.right_shift(cv, 16)
                p_in = inc & jnp.int32(0xFFFF)
                c_in = jnp.right_shift(inc, 16)
                num = ((c_in - khi).astype(jnp.float32)
                       + khi.astype(jnp.float32) * jnp.float32(0.5)
                       + jnp.float32(0.5))
                den = ((p_in - nlo).astype(jnp.float32)
                       + nlo.astype(jnp.float32) * jnp.float32(0.5)
                       + jnp.float32(0.5))
                contrib = num / den
                return acc + jnp.where(t == 1, contrib, jnp.float32(0.0))
            acc = lax.fori_loop(0, NVEC, p2, jnp.zeros((16,), jnp.float32))
            zf16 = jnp.zeros((16,), jnp.float32)
            num_v = jnp.sum(acc) + zf16
            den_v = pcount.astype(jnp.float32) + jnp.float32(EPS) + zf16
            plsc.store_scatter(apbuf, [jnp.zeros((16,), jnp.int32) + lr],
                               num_v / den_v, mask=iota16 == 0)
            return c
        lax.fori_loop(0, rows_per, row_body, 0)
        pltpu.sync_copy(apbuf, ap_hbm.at[pl.ds(wid * rows_per, rows_per)])

    return sc_ap


_sc_ap = _make_sc_kernel()


def _tc_finish(ap_ref, o_ref):
    o_ref[0, 0] = jnp.float32(1.0) - jnp.sum(ap_ref[...]) / jnp.float32(R)


def kernel(output, target):
    ap = _sc_ap(output, target.astype(jnp.int32))
    res = pl.pallas_call(
        _tc_finish,
        out_shape=jax.ShapeDtypeStruct((1, 1), jnp.float32),
        out_specs=pl.BlockSpec(memory_space=pltpu.SMEM),
    )(ap.reshape(16, 128))
    return res[0, 0]
